# baseline (speedup 1.0000x reference)
import functools
import math

import jax
import jax.numpy as jnp
from jax import lax
from jax.experimental import pallas as pl
from jax.experimental.pallas import tpu as pltpu
from jax.experimental.pallas import tpu_sc as plsc

EMB = 128
SCALE = math.sqrt(128.0)

NC = 2
NS = 16
NW = NC * NS
CH = 64
LANES = 16
NBUF = 10
RU = 2


@functools.cache
def _build(B):
    assert B % (NW * CH) == 0
    nchunk = B // (NW * CH)
    assert nchunk % NBUF == 0
    b_per_w = nchunk * CH
    mesh = plsc.VectorSubcoreMesh(core_axis_name="c", subcore_axis_name="s")

    @functools.partial(
        pl.kernel,
        mesh=mesh,
        out_type=jax.ShapeDtypeStruct((B, EMB), jnp.float32),
        compiler_params=pltpu.CompilerParams(needs_layout_passes=False),
        scratch_types=[
            pltpu.VMEM((nchunk, CH), jnp.int32),
        ]
        + [pltpu.VMEM((CH, EMB), jnp.float32) for _ in range(NBUF)]
        + [pltpu.SemaphoreType.DMA for _ in range(NBUF)],
    )
    def emb_kernel(idx_hbm, table_hbm, out_hbm, idx_v, *bufs_sems):
        bufs = bufs_sems[:NBUF]
        sems = bufs_sems[NBUF:]
        wid = lax.axis_index("s") * NC + lax.axis_index("c")
        base = wid * b_per_w
        pltpu.sync_copy(idx_hbm.at[wid], idx_v)

        def fire_chunk(cc, b):
            def fire_group(g, carry):
                vec = idx_v[cc, pl.ds(g * LANES, LANES)]
                for k in range(LANES):
                    pltpu.async_copy(
                        table_hbm.at[vec[k]], bufs[b].at[g * LANES + k], sems[b]
                    )
                return carry

            lax.fori_loop(0, CH // LANES, fire_group, 0)

        def drain_chunk(b):
            pltpu.make_async_copy(
                table_hbm.at[pl.ds(0, CH)], bufs[b], sems[b]
            ).wait()

        for b in range(NBUF):
            fire_chunk(b, b)

        def outer_body(o, carry):
            for b in range(NBUF):
                cc = o * NBUF + b
                drain_chunk(b)

                def row_body(r, carry2):
                    for rr in range(RU):
                        for j in range(EMB // LANES):
                            sl = pl.ds(j * LANES, LANES)
                            bufs[b][r * RU + rr, sl] = bufs[b][r * RU + rr, sl] * SCALE
                    return carry2

                lax.fori_loop(0, CH // RU, row_body, 0)
                pltpu.sync_copy(bufs[b], out_hbm.at[pl.ds(base + cc * CH, CH)])
                nxt = cc + NBUF

                @pl.when(nxt < nchunk)
                def _():
                    fire_chunk(nxt, b)

            return carry

        lax.fori_loop(0, nchunk // NBUF, outer_body, 0)

    return emb_kernel


def kernel(x, table):
    s0, s1 = x.shape
    B = s0 * s1
    idx = x.reshape(NW, B // (NW * CH), CH).astype(jnp.int32)
    out = _build(B)(idx, table)
    return out.reshape(s0, s1, EMB)


# speedup vs baseline: 1.0132x; 1.0132x over previous
import functools
import math

import jax
import jax.numpy as jnp
from jax import lax
from jax.experimental import pallas as pl
from jax.experimental.pallas import tpu as pltpu
from jax.experimental.pallas import tpu_sc as plsc

EMB = 128
SCALE = math.sqrt(128.0)

NC = 2
NS = 16
NW = NC * NS
CH = 64
LANES = 16
NBUF = 10
RU = 2


@functools.cache
def _build(B):
    assert B % (NW * CH) == 0
    nchunk = B // (NW * CH)
    assert nchunk % NBUF == 0
    b_per_w = nchunk * CH
    mesh = plsc.VectorSubcoreMesh(core_axis_name="c", subcore_axis_name="s")

    @functools.partial(
        pl.kernel,
        mesh=mesh,
        out_type=jax.ShapeDtypeStruct((B, EMB), jnp.float32),
        compiler_params=pltpu.CompilerParams(needs_layout_passes=False),
        scratch_types=[
            pltpu.VMEM((nchunk, CH), jnp.int32),
        ]
        + [pltpu.VMEM((CH, EMB), jnp.float32) for _ in range(NBUF)]
        + [pltpu.SemaphoreType.DMA for _ in range(NBUF)]
        + [pltpu.SemaphoreType.DMA for _ in range(NBUF)],
    )
    def emb_kernel(idx_hbm, table_hbm, out_hbm, idx_v, *bufs_sems):
        bufs = bufs_sems[:NBUF]
        sems = bufs_sems[NBUF:2 * NBUF]
        dsems = bufs_sems[2 * NBUF:]
        HALF = CH // 2
        wid = lax.axis_index("s") * NC + lax.axis_index("c")
        base = wid * b_per_w
        pltpu.sync_copy(idx_hbm.at[wid], idx_v)

        def fire_chunk(cc, b):
            pltpu.async_copy(
                table_hbm.at[idx_v.at[cc, pl.ds(0, HALF)]],
                bufs[b].at[pl.ds(0, HALF)],
                sems[b],
            )

            def fire_group(g, carry):
                vec = idx_v[cc, pl.ds(HALF + g * LANES, LANES)]
                for k in range(LANES):
                    pltpu.async_copy(
                        table_hbm.at[vec[k]],
                        bufs[b].at[HALF + g * LANES + k],
                        dsems[b],
                    )
                return carry

            lax.fori_loop(0, HALF // LANES, fire_group, 0)

        def drain_chunk(cc, b):
            pltpu.make_async_copy(
                table_hbm.at[idx_v.at[cc, pl.ds(0, HALF)]],
                bufs[b].at[pl.ds(0, HALF)],
                sems[b],
            ).wait()
            pltpu.make_async_copy(
                table_hbm.at[pl.ds(0, HALF)], bufs[b].at[pl.ds(HALF, HALF)], dsems[b]
            ).wait()

        for b in range(NBUF):
            fire_chunk(b, b)

        def outer_body(o, carry):
            for b in range(NBUF):
                cc = o * NBUF + b
                drain_chunk(cc, b)

                def row_body(r, carry2):
                    for rr in range(RU):
                        for j in range(EMB // LANES):
                            sl = pl.ds(j * LANES, LANES)
                            bufs[b][r * RU + rr, sl] = bufs[b][r * RU + rr, sl] * SCALE
                    return carry2

                lax.fori_loop(0, CH // RU, row_body, 0)
                pltpu.sync_copy(bufs[b], out_hbm.at[pl.ds(base + cc * CH, CH)])
                nxt = cc + NBUF

                @pl.when(nxt < nchunk)
                def _():
                    fire_chunk(nxt, b)

            return carry

        lax.fori_loop(0, nchunk // NBUF, outer_body, 0)

    return emb_kernel


def kernel(x, table):
    s0, s1 = x.shape
    B = s0 * s1
    idx = x.reshape(NW, B // (NW * CH), CH).astype(jnp.int32)
    out = _build(B)(idx, table)
    return out.reshape(s0, s1, EMB)


# speedup vs baseline: 1.0344x; 1.0210x over previous
import functools
import math

import jax
import jax.numpy as jnp
from jax import lax
from jax.experimental import pallas as pl
from jax.experimental.pallas import tpu as pltpu
from jax.experimental.pallas import tpu_sc as plsc

EMB = 128
SCALE = math.sqrt(128.0)

NC = 2
NS = 16
NW = NC * NS
CH = 64
LANES = 16
NBUF = 5
NOB = 5
RU = 2


@functools.cache
def _build(B):
    assert B % (NW * CH) == 0
    nchunk = B // (NW * CH)
    assert nchunk % NBUF == 0 and NBUF % NOB == 0
    b_per_w = nchunk * CH
    mesh = plsc.VectorSubcoreMesh(core_axis_name="c", subcore_axis_name="s")

    @functools.partial(
        pl.kernel,
        mesh=mesh,
        out_type=jax.ShapeDtypeStruct((B, EMB), jnp.float32),
        scratch_types=[
            pltpu.VMEM((nchunk, CH), jnp.int32),
        ]
        + [pltpu.VMEM((CH, EMB), jnp.float32) for _ in range(NBUF)]
        + [pltpu.VMEM((CH, EMB), jnp.float32) for _ in range(NOB)]
        + [pltpu.SemaphoreType.DMA for _ in range(NBUF)]
        + [pltpu.SemaphoreType.DMA for _ in range(NOB)],
    )
    def emb_kernel(idx_hbm, table_hbm, out_hbm, idx_v, *scratch):
        bufs = scratch[:NBUF]
        obufs = scratch[NBUF:NBUF + NOB]
        gsems = scratch[NBUF + NOB:2 * NBUF + NOB]
        osems = scratch[2 * NBUF + NOB:]
        wid = lax.axis_index("s") * NC + lax.axis_index("c")
        base = wid * b_per_w
        pltpu.sync_copy(idx_hbm.at[wid], idx_v)

        for b in range(NBUF):
            pltpu.async_copy(table_hbm.at[idx_v.at[b]], bufs[b], gsems[b])

        def outer_body(o, carry):
            for b in range(NBUF):
                ob = b % NOB
                cc = o * NBUF + b
                pltpu.make_async_copy(
                    table_hbm.at[idx_v.at[cc]], bufs[b], gsems[b]
                ).wait()

                @pl.when(cc >= NOB)
                def _():
                    pltpu.make_async_copy(
                        obufs[ob], out_hbm.at[pl.ds(base, CH)], osems[ob]
                    ).wait()

                def row_body(r, carry2):
                    for rr in range(RU):
                        row = r * RU + rr
                        for j in range(EMB // LANES):
                            sl = pl.ds(j * LANES, LANES)
                            obufs[ob][row, sl] = bufs[b][row, sl] * SCALE
                    return carry2

                lax.fori_loop(0, CH // RU, row_body, 0)

                nxt = cc + NBUF

                @pl.when(nxt < nchunk)
                def _():
                    pltpu.async_copy(table_hbm.at[idx_v.at[nxt]], bufs[b], gsems[b])

                pltpu.async_copy(
                    obufs[ob], out_hbm.at[pl.ds(base + cc * CH, CH)], osems[ob]
                )

            return carry

        lax.fori_loop(0, nchunk // NBUF, outer_body, 0)

        for ob in range(NOB):
            pltpu.make_async_copy(
                obufs[ob], out_hbm.at[pl.ds(base, CH)], osems[ob]
            ).wait()

    return emb_kernel


def kernel(x, table):
    s0, s1 = x.shape
    B = s0 * s1
    idx = x.reshape(NW, B // (NW * CH), CH).astype(jnp.int32)
    out = _build(B)(idx, table)
    return out.reshape(s0, s1, EMB)


# speedup vs baseline: 1.8588x; 1.7969x over previous
import functools
import math

import jax
import jax.numpy as jnp
from jax import lax
from jax.experimental import pallas as pl
from jax.experimental.pallas import tpu as pltpu
from jax.experimental.pallas import tpu_sc as plsc

EMB = 128
SCALE = math.sqrt(128.0)

NC = 2
NS = 16
NW = NC * NS
LANES = 16
NBUF = 8
NOB = 4
RU = 2


@functools.cache
def _build(S0, S1):
    assert S0 % NW == 0 and S1 % RU == 0
    nchunk = S0 // NW
    assert nchunk % NBUF == 0 and NBUF % NOB == 0
    mesh = plsc.VectorSubcoreMesh(core_axis_name="c", subcore_axis_name="s")

    @functools.partial(
        pl.kernel,
        mesh=mesh,
        out_type=jax.ShapeDtypeStruct((S0, S1, EMB), jnp.float32),
        scratch_types=[
            pltpu.VMEM((nchunk, S1), jnp.int32),
        ]
        + [pltpu.VMEM((S1, EMB), jnp.float32) for _ in range(NBUF)]
        + [pltpu.VMEM((S1, EMB), jnp.float32) for _ in range(NOB)]
        + [pltpu.SemaphoreType.DMA for _ in range(NBUF)]
        + [pltpu.SemaphoreType.DMA for _ in range(NOB)],
    )
    def emb_kernel(idx_hbm, table_hbm, out_hbm, idx_v, *scratch):
        bufs = scratch[:NBUF]
        obufs = scratch[NBUF:NBUF + NOB]
        gsems = scratch[NBUF + NOB:2 * NBUF + NOB]
        osems = scratch[2 * NBUF + NOB:]
        wid = lax.axis_index("s") * NC + lax.axis_index("c")
        base = wid * nchunk
        pltpu.sync_copy(idx_hbm.at[pl.ds(base, nchunk)], idx_v)

        for b in range(NBUF):
            pltpu.async_copy(table_hbm.at[idx_v.at[b]], bufs[b], gsems[b])

        def outer_body(o, carry):
            for b in range(NBUF):
                ob = b % NOB
                cc = o * NBUF + b
                pltpu.make_async_copy(
                    table_hbm.at[idx_v.at[cc]], bufs[b], gsems[b]
                ).wait()

                @pl.when(cc >= NOB)
                def _():
                    pltpu.make_async_copy(
                        obufs[ob], out_hbm.at[base], osems[ob]
                    ).wait()

                def row_body(r, carry2):
                    for rr in range(RU):
                        row = r * RU + rr
                        for j in range(EMB // LANES):
                            sl = pl.ds(j * LANES, LANES)
                            obufs[ob][row, sl] = bufs[b][row, sl] * SCALE
                    return carry2

                lax.fori_loop(0, S1 // RU, row_body, 0)

                nxt = cc + NBUF

                @pl.when(nxt < nchunk)
                def _():
                    pltpu.async_copy(table_hbm.at[idx_v.at[nxt]], bufs[b], gsems[b])

                pltpu.async_copy(obufs[ob], out_hbm.at[base + cc], osems[ob])

            return carry

        lax.fori_loop(0, nchunk // NBUF, outer_body, 0)

        for ob in range(NOB):
            pltpu.make_async_copy(
                obufs[ob], out_hbm.at[base], osems[ob]
            ).wait()

    return emb_kernel


def kernel(x, table):
    s0, s1 = x.shape
    return _build(s0, s1)(x.astype(jnp.int32), table)


# speedup vs baseline: 1.8597x; 1.0005x over previous
import functools
import math

import jax
import jax.numpy as jnp
from jax import lax
from jax.experimental import pallas as pl
from jax.experimental.pallas import tpu as pltpu
from jax.experimental.pallas import tpu_sc as plsc

EMB = 128
SCALE = math.sqrt(128.0)

NC = 2
NS = 16
NW = NC * NS
LANES = 16
NBUF = 8
NOB = 4
RU = 2


@functools.cache
def _build(S0, S1):
    assert S0 % NW == 0 and S1 % RU == 0
    nchunk = S0 // NW
    assert nchunk % NBUF == 0 and NBUF % NOB == 0
    mesh = plsc.VectorSubcoreMesh(core_axis_name="c", subcore_axis_name="s")

    @functools.partial(
        pl.kernel,
        mesh=mesh,
        out_type=jax.ShapeDtypeStruct((S0, S1, EMB), jnp.float32),
        compiler_params=pltpu.CompilerParams(use_tc_tiling_on_sc=True),
        scratch_types=[
            pltpu.VMEM((nchunk, S1), jnp.int32),
        ]
        + [pltpu.VMEM((S1, EMB), jnp.float32) for _ in range(NBUF)]
        + [pltpu.VMEM((S1, EMB), jnp.float32) for _ in range(NOB)]
        + [pltpu.SemaphoreType.DMA for _ in range(NBUF)]
        + [pltpu.SemaphoreType.DMA for _ in range(NOB)],
    )
    def emb_kernel(idx_hbm, table_hbm, out_hbm, idx_v, *scratch):
        bufs = scratch[:NBUF]
        obufs = scratch[NBUF:NBUF + NOB]
        gsems = scratch[NBUF + NOB:2 * NBUF + NOB]
        osems = scratch[2 * NBUF + NOB:]
        wid = lax.axis_index("s") * NC + lax.axis_index("c")
        base = wid * nchunk
        pltpu.sync_copy(idx_hbm.at[pl.ds(base, nchunk)], idx_v)

        for b in range(NBUF):
            pltpu.async_copy(table_hbm.at[idx_v.at[b]], bufs[b], gsems[b])

        def outer_body(o, carry):
            for b in range(NBUF):
                ob = b % NOB
                cc = o * NBUF + b
                pltpu.make_async_copy(
                    table_hbm.at[idx_v.at[cc]], bufs[b], gsems[b]
                ).wait()

                @pl.when(cc >= NOB)
                def _():
                    pltpu.make_async_copy(
                        obufs[ob], out_hbm.at[base], osems[ob]
                    ).wait()

                def row_body(r, carry2):
                    for rr in range(RU):
                        row = r * RU + rr
                        for j in range(EMB // LANES):
                            sl = pl.ds(j * LANES, LANES)
                            obufs[ob][row, sl] = bufs[b][row, sl] * SCALE
                    return carry2

                lax.fori_loop(0, S1 // RU, row_body, 0)

                nxt = cc + NBUF

                @pl.when(nxt < nchunk)
                def _():
                    pltpu.async_copy(table_hbm.at[idx_v.at[nxt]], bufs[b], gsems[b])

                pltpu.async_copy(obufs[ob], out_hbm.at[base + cc], osems[ob])

            return carry

        lax.fori_loop(0, nchunk // NBUF, outer_body, 0)

        for ob in range(NOB):
            pltpu.make_async_copy(
                obufs[ob], out_hbm.at[base], osems[ob]
            ).wait()

    return emb_kernel


def kernel(x, table):
    s0, s1 = x.shape
    return _build(s0, s1)(x.astype(jnp.int32), table)
